# baseline (speedup 1.0000x reference)
import functools

import jax
import jax.numpy as jnp
from jax import lax
from jax.experimental import pallas as pl
from jax.experimental.pallas import tpu as pltpu
from jax.experimental.pallas import tpu_sc as plsc

BATCH = 4096
SEQ = 200
EMBED_DIM = 64
PAD_DIM = 128
ROWS = BATCH * SEQ
HALF = ROWS // 2
NUM_CORES = 2
NUM_SUBCORES = 16
NW = NUM_CORES * NUM_SUBCORES
RPW = HALF // NW
CHUNK = 128
NCHUNK = RPW // CHUNK
NPAIR = NCHUNK // 2
LANES = 16
CPR = EMBED_DIM // LANES
RB = 512


def _sc_body(tab, idx, pos, L2, pos_v,
             idxa0_v, idxb0_v, idxa1_v, idxb1_v,
             rowsa0_v, rowsb0_v, rowsa1_v, rowsb1_v,
             stg0_v, stg1_v,
             gsem0, gsem1, osem0, osem1, isem0, isem1):
    wid = lax.axis_index("s") * NUM_CORES + lax.axis_index("c")
    base = wid * RPW
    idxa_vs = (idxa0_v, idxa1_v)
    idxb_vs = (idxb0_v, idxb1_v)
    rowsa_vs = (rowsa0_v, rowsa1_v)
    rowsb_vs = (rowsb0_v, rowsb1_v)
    stg_vs = (stg0_v, stg1_v)
    gsems = (gsem0, gsem1)
    osems = (osem0, osem1)
    isems = (isem0, isem1)

    pltpu.sync_copy(pos, pos_v)

    def start_idx(c, b):
        pltpu.async_copy(
            idx.at[pl.ds(base + c * CHUNK, CHUNK)], idxa_vs[b], isems[b])
        pltpu.async_copy(
            idx.at[pl.ds(HALF + base + c * CHUNK, CHUNK)], idxb_vs[b],
            isems[b])

    def wait_idx(c, b):
        pltpu.make_async_copy(
            idx.at[pl.ds(base + c * CHUNK, CHUNK)], idxa_vs[b],
            isems[b]).wait()
        pltpu.make_async_copy(
            idx.at[pl.ds(HALF + base + c * CHUNK, CHUNK)], idxb_vs[b],
            isems[b]).wait()

    def start_gathers(b):
        pltpu.async_copy(tab.at[idxa_vs[b]], rowsa_vs[b], gsems[b])
        pltpu.async_copy(tab.at[idxb_vs[b]], rowsb_vs[b], gsems[b])

    def wait_gathers(b):
        pltpu.make_async_copy(
            tab.at[pl.ds(0, CHUNK)], rowsa_vs[b], gsems[b]).wait()
        pltpu.make_async_copy(
            tab.at[pl.ds(0, CHUNK)], rowsb_vs[b], gsems[b]).wait()

    def start_out(c, b):
        pltpu.async_copy(
            stg_vs[b], L2.at[pl.ds(base + c * CHUNK, CHUNK)], osems[b])

    def wait_out(c, b):
        pltpu.make_async_copy(
            stg_vs[b], L2.at[pl.ds(base + c * CHUNK, CHUNK)], osems[b]).wait()

    def add_pos(c, b):
        rowsa_v = rowsa_vs[b]
        rowsb_v = rowsb_vs[b]
        stg_v = stg_vs[b]
        s0 = lax.rem(c * CHUNK, SEQ)

        def row_body(r, rp):
            for cc in range(CPR):
                sl = pl.ds(cc * LANES, LANES)
                p = pos_v[rp, sl]
                stg_v[r, sl] = rowsa_v[r, sl] + p
                stg_v[r, pl.ds(EMBED_DIM + cc * LANES, LANES)] = (
                    rowsb_v[r, sl] + p)
            nrp = rp + 1
            return lax.select(nrp == SEQ, 0, nrp)

        lax.fori_loop(0, CHUNK, row_body, s0)

    for b in range(2):
        start_idx(b, b)
    for b in range(2):
        wait_idx(b, b)
        start_gathers(b)

    def pair_body(k, _):
        for b in range(2):
            c = 2 * k + b
            wait_gathers(b)

            @pl.when(k < NPAIR - 1)
            def _(c=c, b=b):
                start_idx(c + 2, b)

            @pl.when(k > 0)
            def _(c=c, b=b):
                wait_out(c - 2, b)

            add_pos(c, b)
            start_out(c, b)

            @pl.when(k < NPAIR - 1)
            def _(c=c, b=b):
                wait_idx(c + 2, b)
                start_gathers(b)

        return 0

    lax.fori_loop(0, NPAIR, pair_body, 0)
    wait_out(NCHUNK - 2, 0)
    wait_out(NCHUNK - 1, 1)


def _depad_body(l_ref, o_ref):
    j = pl.program_id(1)

    @pl.when(j == 0)
    def _():
        o_ref[...] = l_ref[:, :EMBED_DIM]

    @pl.when(j == 1)
    def _():
        o_ref[...] = l_ref[:, EMBED_DIM:]


def _branch(tab, idx, pos):
    mesh = plsc.VectorSubcoreMesh(core_axis_name="c", subcore_axis_name="s")
    gather = functools.partial(
        pl.kernel,
        mesh=mesh,
        out_type=jax.ShapeDtypeStruct((HALF, PAD_DIM), jnp.float32),
        scratch_types=[
            pltpu.VMEM((SEQ, EMBED_DIM), jnp.float32),
        ] + [pltpu.VMEM((CHUNK,), jnp.int32)] * 4
          + [pltpu.VMEM((CHUNK, PAD_DIM), jnp.float32)] * 4
          + [pltpu.VMEM((CHUNK, PAD_DIM), jnp.float32)] * 2
          + [pltpu.SemaphoreType.DMA] * 6,
    )(_sc_body)
    L2 = gather(tab, idx, pos)
    out = pl.pallas_call(
        _depad_body,
        grid=(HALF // RB, 2),
        in_specs=[pl.BlockSpec((RB, PAD_DIM), lambda i, j: (i, 0))],
        out_specs=pl.BlockSpec(
            (RB, EMBED_DIM), lambda i, j: (j * (HALF // RB) + i, 0)),
        out_shape=jax.ShapeDtypeStruct((ROWS, EMBED_DIM), jnp.float32),
    )(L2)
    return out.reshape(BATCH, SEQ, EMBED_DIM)


@jax.jit
def kernel(g_tok_table, e_tok_table, g_pos_table, e_pos_table,
           g_text_tokens, e_text_tokens):
    g_idx = g_text_tokens.reshape(ROWS).astype(jnp.int32)
    e_idx = e_text_tokens.reshape(ROWS).astype(jnp.int32)
    padc = ((0, 0), (0, PAD_DIM - EMBED_DIM))
    g_tab = jnp.pad(g_tok_table, padc)
    e_tab = jnp.pad(e_tok_table, padc)
    g_out = _branch(g_tab, g_idx, g_pos_table)
    e_out = _branch(e_tab, e_idx, e_pos_table)
    return (g_out, e_out)
